# baseline (speedup 1.0000x reference)
import functools

import jax
import jax.numpy as jnp
from jax import lax
from jax.experimental import pallas as pl
from jax.experimental.pallas import tpu as pltpu
from jax.experimental.pallas import tpu_sc as plsc

N = 10000
E = 320000
F = 64
ALPHA = 0.5
N_LAYERS = 3

NC = 2
NS = 16
NW = NC * NS
EPW = E // NW
SUB = 80
ROWS = E // SUB
RPW = ROWS // NW
CH = 400
SPC = CH // SUB
NCHUNK = EPW // CH
NPAD = 10240
NPT = NPAD // NS


def _pad_to(a, shape):
    pads = [(0, t - s) for s, t in zip(a.shape, shape)]
    return jnp.pad(a, pads)


def _bd(w):
    k, m = w.shape
    z = jnp.zeros((k, m), w.dtype)
    return jnp.concatenate(
        [jnp.concatenate([w, z], axis=1), jnp.concatenate([z, w], axis=1)],
        axis=0)


def _dup(b):
    return jnp.concatenate([b, b]).reshape(1, -1)


def _dot(a, b):
    return jax.lax.dot_general(a, b, (((1,), (0,)), ((), ())),
                               preferred_element_type=jnp.float32)


_NBLK = 2000
_PBLK = 3200
EP = E // 2


def _proj_body(x_ref, ws_ref, wd_ref, ps_ref, pd_ref):
    x = x_ref[...]
    ps_ref[...] = _dot(x, ws_ref[...])
    pd_ref[...] = _dot(x, wd_ref[...])


def _proj0(x, wsp, wdp):
    nd = x.shape[1]
    return pl.pallas_call(
        _proj_body,
        out_shape=(jax.ShapeDtypeStruct((N, F), jnp.float32),
                   jax.ShapeDtypeStruct((N, F), jnp.float32)),
        grid=(N // _NBLK,),
        in_specs=[pl.BlockSpec((_NBLK, nd), lambda i: (i, 0)),
                  pl.BlockSpec((nd, F), lambda i: (0, 0)),
                  pl.BlockSpec((nd, F), lambda i: (0, 0))],
        out_specs=(pl.BlockSpec((_NBLK, F), lambda i: (i, 0)),
                   pl.BlockSpec((_NBLK, F), lambda i: (i, 0))),
    )(x, wsp, wdp)


def _qe0_body(ea_a_ref, ea_b_ref, we_ref, b_ref, qe_ref):
    we = we_ref[...]
    b = b_ref[...]
    qa = _dot(ea_a_ref[...], we) + b
    qb = _dot(ea_b_ref[...], we) + b
    qe_ref[...] = jnp.concatenate([qa, qb], axis=1)


def _qe0(ea, wep, b1p):
    k = ea.shape[1]
    nblk = EP // _PBLK
    return pl.pallas_call(
        _qe0_body,
        out_shape=jax.ShapeDtypeStruct((EP, 2 * F), jnp.float32),
        grid=(nblk,),
        in_specs=[pl.BlockSpec((_PBLK, k), lambda i: (i, 0)),
                  pl.BlockSpec((_PBLK, k), lambda i: (i + nblk, 0)),
                  pl.BlockSpec((k, F), lambda i: (0, 0)),
                  pl.BlockSpec((1, F), lambda i: (0, 0))],
        out_specs=pl.BlockSpec((_PBLK, 2 * F), lambda i: (i, 0)),
    )(ea, ea, wep, b1p)


def _eot_body(nblk, w, eo_ref, out_ref):
    x = eo_ref[...]
    half = pl.program_id(0) // nblk
    sel = jnp.where(half == 0, x[:, :F], x[:, F:])
    out_ref[...] = sel.T[:w]


def _eot(eo_p, width):
    w = max(8, width)
    nblk = EP // _PBLK
    return pl.pallas_call(
        functools.partial(_eot_body, nblk, w),
        out_shape=jax.ShapeDtypeStruct((w, E), jnp.float32),
        grid=(2 * nblk,),
        in_specs=[pl.BlockSpec((_PBLK, 2 * F), lambda i: (i % nblk, 0))],
        out_specs=pl.BlockSpec((w, _PBLK), lambda i: (0, i)),
    )(eo_p)


def _edge_body(r_ref, w2_ref, b2_ref, wen_ref, bn_ref, eo_ref, qn_ref):
    eo = _dot(r_ref[...], w2_ref[...]) + b2_ref[...]
    eo_ref[...] = eo
    qn_ref[...] = _dot(eo, wen_ref[...]) + bn_ref[...]


def _edge_tc(r_p, w2_bd, b2_p, wen_bd, bn_p):
    return pl.pallas_call(
        _edge_body,
        out_shape=(jax.ShapeDtypeStruct((EP, 2 * F), jnp.float32),
                   jax.ShapeDtypeStruct((EP, 2 * F), jnp.float32)),
        grid=(EP // _PBLK,),
        in_specs=[pl.BlockSpec((_PBLK, 2 * F), lambda i: (i, 0)),
                  pl.BlockSpec((2 * F, 2 * F), lambda i: (0, 0)),
                  pl.BlockSpec((1, 2 * F), lambda i: (0, 0)),
                  pl.BlockSpec((2 * F, 2 * F), lambda i: (0, 0)),
                  pl.BlockSpec((1, 2 * F), lambda i: (0, 0))],
        out_specs=(pl.BlockSpec((_PBLK, 2 * F), lambda i: (i, 0)),
                   pl.BlockSpec((_PBLK, 2 * F), lambda i: (i, 0))),
    )(r_p, w2_bd, b2_p, wen_bd, bn_p)


def _edge_last_body(r_ref, w2_ref, b2_ref, eo_ref):
    eo_ref[...] = _dot(r_ref[...], w2_ref[...]) + b2_ref[...]


def _edge_tc_last(r_p, w2_bd, b2_p):
    return pl.pallas_call(
        _edge_last_body,
        out_shape=jax.ShapeDtypeStruct((EP, 2 * F), jnp.float32),
        grid=(EP // _PBLK,),
        in_specs=[pl.BlockSpec((_PBLK, 2 * F), lambda i: (i, 0)),
                  pl.BlockSpec((2 * F, 2 * F), lambda i: (0, 0)),
                  pl.BlockSpec((1, 2 * F), lambda i: (0, 0))],
        out_specs=pl.BlockSpec((_PBLK, 2 * F), lambda i: (i, 0)),
    )(r_p, w2_bd, b2_p)


def _node_body(has_res, has_next, refs):
    (x_ref, a0_ref, a1_ref, woa_ref, wox_ref, bo1_ref,
     wo2_ref, bo2_ref) = refs[:8]
    idx = 8
    if has_res:
        r1w_ref, r1b_ref, r2w_ref, r2b_ref = refs[idx:idx + 4]
        idx += 4
    if has_next:
        wsn_ref, wdn_ref = refs[idx:idx + 2]
        idx += 2
    outs = refs[idx:]
    x = x_ref[...]
    agg = a0_ref[...] + a1_ref[...]
    h = _dot(x, wox_ref[...]) + _dot(agg, woa_ref[...]) + bo1_ref[...]
    h = jnp.maximum(h, 0.0)
    delta = _dot(h, wo2_ref[...]) + bo2_ref[...]
    if has_res:
        hr = jnp.maximum(_dot(x, r1w_ref[...]) + r1b_ref[...], 0.0)
        res = jnp.maximum(_dot(hr, r2w_ref[...]) + r2b_ref[...], 0.0)
    else:
        res = x
    xn = ALPHA * res + (1.0 - ALPHA) * jnp.maximum(delta, 0.0)
    outs[0][...] = xn
    if has_next:
        outs[1][...] = _dot(xn, wsn_ref[...])
        outs[2][...] = _dot(xn, wdn_ref[...])


def _node_tc(x, a0, a1, woa, wox, bo1, wo2, bo2, res_w=None, next_w=None):
    ndo = wo2.shape[1]
    has_res = res_w is not None
    has_next = next_w is not None
    args = [x, a0, a1, woa, wox, bo1, wo2, bo2]
    if has_res:
        args += list(res_w)
    if has_next:
        args += list(next_w)
    out_shape = [jax.ShapeDtypeStruct((N, ndo), jnp.float32)]
    if has_next:
        out_shape += [jax.ShapeDtypeStruct((N, F), jnp.float32),
                      jax.ShapeDtypeStruct((N, F), jnp.float32)]

    def body(*refs):
        _node_body(has_res, has_next, refs)

    n_row_args = 3
    in_specs = [pl.BlockSpec((_NBLK, a.shape[1]), lambda i: (i, 0))
                for a in args[:n_row_args]]
    in_specs += [pl.BlockSpec(a.shape, lambda i: (0, 0))
                 for a in args[n_row_args:]]
    out_specs = tuple(pl.BlockSpec((_NBLK, s.shape[1]), lambda i: (i, 0))
                      for s in out_shape)
    outs = pl.pallas_call(
        body,
        out_shape=tuple(out_shape),
        grid=(N // _NBLK,),
        in_specs=in_specs,
        out_specs=out_specs,
    )(*args)
    return outs


def _gather_body(ps_hbm, pd_hbm, qe_hbm, src_hbm, dst_hbm, r_hbm,
                 srcv, dstv, ga, gb, qv, sem):
    c = lax.axis_index("c")
    s = lax.axis_index("s")
    wid = s * NC + c
    half = wid // NS
    jbase0 = (wid % NS) * EPW

    pltpu.sync_copy(src_hbm.at[wid], srcv)
    pltpu.sync_copy(dst_hbm.at[wid], dstv)

    def chunk(ci, _):
        jbase = jbase0 + ci * CH
        irow = ci * SPC
        pltpu.sync_copy(qe_hbm.at[pl.ds(jbase, CH), half], qv)
        cps = []
        for j in range(SPC):
            cps.append(pltpu.async_copy(
                ps_hbm.at[srcv.at[irow + j]],
                ga.at[pl.ds(j * SUB, SUB)], sem))
            cps.append(pltpu.async_copy(
                pd_hbm.at[dstv.at[irow + j]],
                gb.at[pl.ds(j * SUB, SUB)], sem))
        for cp in cps:
            cp.wait()

        def row(rj, _):
            for col in (0, 16, 32, 48):
                a = ga[rj, pl.ds(col, 16)]
                b = gb[rj, pl.ds(col, 16)]
                q = qv[rj, pl.ds(col, 16)]
                ga[rj, pl.ds(col, 16)] = jnp.maximum(a + b + q, 0.0)
            return 0
        lax.fori_loop(0, CH, row, 0)

        pltpu.sync_copy(ga, r_hbm.at[pl.ds(jbase, CH), half])
        return 0
    lax.fori_loop(0, NCHUNK, chunk, 0)


@functools.cache
def _get_gather_sc():
    mesh = plsc.VectorSubcoreMesh(core_axis_name="c", subcore_axis_name="s")
    return pl.kernel(
        _gather_body,
        out_type=jax.ShapeDtypeStruct((EP, 2, F), jnp.float32),
        mesh=mesh,
        scratch_types=[
            pltpu.VMEM((RPW, SUB), jnp.int32),
            pltpu.VMEM((RPW, SUB), jnp.int32),
            pltpu.VMEM((CH, F), jnp.float32),
            pltpu.VMEM((CH, F), jnp.float32),
            pltpu.VMEM((CH, F), jnp.float32),
            pltpu.SemaphoreType.DMA,
        ],
        compiler_params=pltpu.CompilerParams(use_tc_tiling_on_sc=False),
    )


def _gather_sc(ps, pd, qe, src, dst):
    return _get_gather_sc()(ps, pd, qe, src, dst)


def _scatter_body(eo_hbm, dst_hbm, apart_hbm, dstv, ev, zv, acc, sem):
    c = lax.axis_index("c")
    s = lax.axis_index("s")
    wid = s * NC + c
    half = wid // NS
    jbase0 = (wid % NS) * EPW

    def fill_zero(j, _):
        for col in (0, 16, 32, 48):
            zv[j, pl.ds(col, 16)] = jnp.zeros((16,), jnp.float32)
        return 0
    lax.fori_loop(0, NPT, fill_zero, 0)
    pltpu.sync_copy(zv, acc.at[pl.ds(s * NPT, NPT)])
    plsc.subcore_barrier()

    pltpu.sync_copy(dst_hbm.at[wid], dstv)

    def chunk(ci, _):
        jbase = jbase0 + ci * CH
        irow = ci * SPC
        pltpu.sync_copy(eo_hbm.at[pl.ds(jbase, CH), half], ev)
        for j in range(SPC):
            pltpu.sync_copy(ev.at[pl.ds(j * SUB, SUB)],
                            acc.at[dstv.at[irow + j]], add=True)
        return 0
    lax.fori_loop(0, NCHUNK, chunk, 0)

    plsc.subcore_barrier()
    pltpu.sync_copy(acc.at[pl.ds(s * NPT, NPT)], zv)
    pltpu.sync_copy(zv, apart_hbm.at[c, pl.ds(s * NPT, NPT)])


@functools.cache
def _get_scatter_sc():
    mesh = plsc.VectorSubcoreMesh(core_axis_name="c", subcore_axis_name="s")
    return pl.kernel(
        _scatter_body,
        out_type=jax.ShapeDtypeStruct((NC, NPAD, F), jnp.float32),
        mesh=mesh,
        scratch_types=[
            pltpu.VMEM((RPW, SUB), jnp.int32),
            pltpu.VMEM((CH, F), jnp.float32),
            pltpu.VMEM((NPT, F), jnp.float32),
            pltpu.VMEM_SHARED((NPAD, F), jnp.float32),
            pltpu.SemaphoreType.DMA,
        ],
        compiler_params=pltpu.CompilerParams(use_tc_tiling_on_sc=False),
    )


def _scatter_sc(eo, dst):
    return _get_scatter_sc()(eo, dst)


def kernel(x, edge_index, edge_attr, params):
    src = edge_index[0].astype(jnp.int32).reshape(NW, RPW, SUB)
    dst = edge_index[1].astype(jnp.int32).reshape(NW, RPW, SUB)

    L = []
    for i in range(N_LAYERS):
        lp = params["layers"][i]
        rp = params["res"][i]
        nd = [128, 64, 64][i]
        w1 = lp["rel"]["l1"]["W"]
        b1 = lp["rel"]["l1"]["b"]
        ws, wd, we = w1[:nd], w1[nd:2 * nd], w1[2 * nd:]
        w2 = lp["rel"]["l2"]["W"]
        b2 = lp["rel"]["l2"]["b"]
        wo1 = lp["obj"]["l1"]["W"]
        bo1 = lp["obj"]["l1"]["b"]
        wox, woa = wo1[:nd], wo1[nd:]
        wo2 = lp["obj"]["l2"]["W"]
        bo2 = lp["obj"]["l2"]["b"]
        L.append(dict(nd=nd, ws=ws, wd=wd, we=we, b1=b1, w2=w2, b2=b2,
                      wox=wox, woa=woa, bo1=bo1, wo2=wo2, bo2=bo2, rp=rp))

    for li in L:
        li["wsp"] = _pad_to(li["ws"], (li["nd"], F))
        li["wdp"] = _pad_to(li["wd"], (li["nd"], F))
        do = li["w2"].shape[1]
        li["w2bd"] = _bd(_pad_to(li["w2"], (F, F)))
        li["b2p"] = _dup(_pad_to(li["b2"], (F,)))
        li["webd"] = _bd(_pad_to(li["we"], (li["we"].shape[0], F)))
        li["b1p"] = _dup(_pad_to(li["b1"], (F,)))
        li["woap"] = _pad_to(li["woa"], (F, 40))

    l2 = L[2]
    wo2_last = _pad_to(l2["wo2"], (40, 8))
    bo2_last = _pad_to(l2["bo2"].reshape(1, 3), (1, 8))
    r1w_last = _pad_to(l2["rp"]["l1"]["W"], (64, 8))
    r1b_last = _pad_to(l2["rp"]["l1"]["b"].reshape(1, 3), (1, 8))
    r2w_last = _pad_to(l2["rp"]["l2"]["W"], (8, 8))
    r2b_last = _pad_to(l2["rp"]["l2"]["b"].reshape(1, 3), (1, 8))

    ps, pd = _proj0(x, L[0]["wsp"], L[0]["wdp"])
    qe_p = _qe0(edge_attr, _pad_to(L[0]["we"], (16, F)),
                _pad_to(L[0]["b1"].reshape(1, 40), (1, F)))

    xs = [x]
    eas = [edge_attr]
    xcur = x
    for i in range(N_LAYERS):
        li = L[i]
        r = _gather_sc(ps, pd, qe_p.reshape(EP, 2, F), src, dst)
        r_p = r.reshape(EP, 2 * F)
        if i < N_LAYERS - 1:
            ln = L[i + 1]
            eo_p, qe_p = _edge_tc(r_p, li["w2bd"], li["b2p"],
                                  ln["webd"], ln["b1p"])
        else:
            eo_p = _edge_tc_last(r_p, li["w2bd"], li["b2p"])
        eot = _eot(eo_p, li["w2"].shape[1])
        eo_leaf = jnp.transpose(eot[:li["w2"].shape[1]])
        apart = _scatter_sc(eo_p.reshape(EP, 2, F), dst)
        a0, a1 = apart[0, :N], apart[1, :N]
        if i < N_LAYERS - 1:
            res_w = None
            if li["rp"] is not None:
                res_w = (li["rp"]["l1"]["W"],
                         li["rp"]["l1"]["b"].reshape(1, -1),
                         li["rp"]["l2"]["W"],
                         li["rp"]["l2"]["b"].reshape(1, -1))
            xcur, ps, pd = _node_tc(
                xcur, a0, a1, li["woap"], li["wox"], li["bo1"].reshape(1, -1),
                li["wo2"], li["bo2"].reshape(1, -1),
                res_w=res_w, next_w=(L[i + 1]["wsp"], L[i + 1]["wdp"]))
        else:
            res_w = (r1w_last, r1b_last, r2w_last, r2b_last)
            (xn8,) = _node_tc(
                xcur, a0, a1, li["woap"], li["wox"], li["bo1"].reshape(1, -1),
                wo2_last, bo2_last, res_w=res_w, next_w=None)
            xcur = xn8[:, :3]
        xs.append(xcur)
        eas.append(eo_leaf)

    return (xcur, tuple(xs), tuple(eas))


# speedup vs baseline: 3.1888x; 3.1888x over previous
import functools

import jax
import jax.numpy as jnp
from jax import lax
from jax.experimental import pallas as pl
from jax.experimental.pallas import tpu as pltpu
from jax.experimental.pallas import tpu_sc as plsc

N = 10000
E = 320000
F = 64
ALPHA = 0.5
N_LAYERS = 3

NC = 2
NS = 16
NW = NC * NS
EPW = E // NW
SUB = 80
ROWS = E // SUB
RPW = ROWS // NW
CH = 400
SPC = CH // SUB
NCHUNK = EPW // CH
NPAD = 10240
NPT = NPAD // NS


def _pad_to(a, shape):
    pads = [(0, t - s) for s, t in zip(a.shape, shape)]
    return jnp.pad(a, pads)


def _bd(w):
    k, m = w.shape
    z = jnp.zeros((k, m), w.dtype)
    return jnp.concatenate(
        [jnp.concatenate([w, z], axis=1), jnp.concatenate([z, w], axis=1)],
        axis=0)


def _dup(b):
    return jnp.concatenate([b, b]).reshape(1, -1)


def _dot(a, b):
    return jax.lax.dot_general(a, b, (((1,), (0,)), ((), ())),
                               preferred_element_type=jnp.float32)


_NBLK = 2000
_PBLK = 3200
EP = E // 2


def _proj_body(x_ref, ws_ref, wd_ref, ps_ref, pd_ref):
    x = x_ref[...]
    ps_ref[...] = _dot(x, ws_ref[...])
    pd_ref[...] = _dot(x, wd_ref[...])


def _proj0(x, wsp, wdp):
    nd = x.shape[1]
    return pl.pallas_call(
        _proj_body,
        out_shape=(jax.ShapeDtypeStruct((N, F), jnp.float32),
                   jax.ShapeDtypeStruct((N, F), jnp.float32)),
        grid=(N // _NBLK,),
        in_specs=[pl.BlockSpec((_NBLK, nd), lambda i: (i, 0)),
                  pl.BlockSpec((nd, F), lambda i: (0, 0)),
                  pl.BlockSpec((nd, F), lambda i: (0, 0))],
        out_specs=(pl.BlockSpec((_NBLK, F), lambda i: (i, 0)),
                   pl.BlockSpec((_NBLK, F), lambda i: (i, 0))),
    )(x, wsp, wdp)


def _qe0_body(ea_a_ref, ea_b_ref, we_ref, b_ref, qe_ref):
    we = we_ref[...]
    b = b_ref[...]
    qa = _dot(ea_a_ref[...], we) + b
    qb = _dot(ea_b_ref[...], we) + b
    qe_ref[...] = jnp.concatenate([qa, qb], axis=1)


def _qe0(ea, wep, b1p):
    k = ea.shape[1]
    nblk = EP // _PBLK
    return pl.pallas_call(
        _qe0_body,
        out_shape=jax.ShapeDtypeStruct((EP, 2 * F), jnp.float32),
        grid=(nblk,),
        in_specs=[pl.BlockSpec((_PBLK, k), lambda i: (i, 0)),
                  pl.BlockSpec((_PBLK, k), lambda i: (i + nblk, 0)),
                  pl.BlockSpec((k, F), lambda i: (0, 0)),
                  pl.BlockSpec((1, F), lambda i: (0, 0))],
        out_specs=pl.BlockSpec((_PBLK, 2 * F), lambda i: (i, 0)),
    )(ea, ea, wep, b1p)


def _eot_body(nblk, w, eo_ref, out_ref):
    x = eo_ref[...]
    half = pl.program_id(0) // nblk
    sel = jnp.where(half == 0, x[:, :F], x[:, F:])
    out_ref[...] = sel.T[:w]


def _eot(eo_p, width):
    w = max(8, width)
    nblk = EP // _PBLK
    return pl.pallas_call(
        functools.partial(_eot_body, nblk, w),
        out_shape=jax.ShapeDtypeStruct((w, E), jnp.float32),
        grid=(2 * nblk,),
        in_specs=[pl.BlockSpec((_PBLK, 2 * F), lambda i: (i % nblk, 0))],
        out_specs=pl.BlockSpec((w, _PBLK), lambda i: (0, i)),
    )(eo_p)


def _edge_body(r_ref, w2_ref, b2_ref, wen_ref, bn_ref, eo_ref, qn_ref):
    eo = _dot(r_ref[...], w2_ref[...]) + b2_ref[...]
    eo_ref[...] = eo
    qn_ref[...] = _dot(eo, wen_ref[...]) + bn_ref[...]


def _edge_tc(r_p, w2_bd, b2_p, wen_bd, bn_p):
    return pl.pallas_call(
        _edge_body,
        out_shape=(jax.ShapeDtypeStruct((EP, 2 * F), jnp.float32),
                   jax.ShapeDtypeStruct((EP, 2 * F), jnp.float32)),
        grid=(EP // _PBLK,),
        in_specs=[pl.BlockSpec((_PBLK, 2 * F), lambda i: (i, 0)),
                  pl.BlockSpec((2 * F, 2 * F), lambda i: (0, 0)),
                  pl.BlockSpec((1, 2 * F), lambda i: (0, 0)),
                  pl.BlockSpec((2 * F, 2 * F), lambda i: (0, 0)),
                  pl.BlockSpec((1, 2 * F), lambda i: (0, 0))],
        out_specs=(pl.BlockSpec((_PBLK, 2 * F), lambda i: (i, 0)),
                   pl.BlockSpec((_PBLK, 2 * F), lambda i: (i, 0))),
    )(r_p, w2_bd, b2_p, wen_bd, bn_p)


def _edge_last_body(r_ref, w2_ref, b2_ref, eo_ref):
    eo_ref[...] = _dot(r_ref[...], w2_ref[...]) + b2_ref[...]


def _edge_tc_last(r_p, w2_bd, b2_p):
    return pl.pallas_call(
        _edge_last_body,
        out_shape=jax.ShapeDtypeStruct((EP, 2 * F), jnp.float32),
        grid=(EP // _PBLK,),
        in_specs=[pl.BlockSpec((_PBLK, 2 * F), lambda i: (i, 0)),
                  pl.BlockSpec((2 * F, 2 * F), lambda i: (0, 0)),
                  pl.BlockSpec((1, 2 * F), lambda i: (0, 0))],
        out_specs=pl.BlockSpec((_PBLK, 2 * F), lambda i: (i, 0)),
    )(r_p, w2_bd, b2_p)


def _node_body(has_res, has_next, refs):
    (x_ref, a0_ref, a1_ref, woa_ref, wox_ref, bo1_ref,
     wo2_ref, bo2_ref) = refs[:8]
    idx = 8
    if has_res:
        r1w_ref, r1b_ref, r2w_ref, r2b_ref = refs[idx:idx + 4]
        idx += 4
    if has_next:
        wsn_ref, wdn_ref = refs[idx:idx + 2]
        idx += 2
    outs = refs[idx:]
    x = x_ref[...]
    agg = a0_ref[...] + a1_ref[...]
    h = _dot(x, wox_ref[...]) + _dot(agg, woa_ref[...]) + bo1_ref[...]
    h = jnp.maximum(h, 0.0)
    delta = _dot(h, wo2_ref[...]) + bo2_ref[...]
    if has_res:
        hr = jnp.maximum(_dot(x, r1w_ref[...]) + r1b_ref[...], 0.0)
        res = jnp.maximum(_dot(hr, r2w_ref[...]) + r2b_ref[...], 0.0)
    else:
        res = x
    xn = ALPHA * res + (1.0 - ALPHA) * jnp.maximum(delta, 0.0)
    outs[0][...] = xn
    if has_next:
        outs[1][...] = _dot(xn, wsn_ref[...])
        outs[2][...] = _dot(xn, wdn_ref[...])


def _node_tc(x, a0, a1, woa, wox, bo1, wo2, bo2, res_w=None, next_w=None):
    ndo = wo2.shape[1]
    has_res = res_w is not None
    has_next = next_w is not None
    args = [x, a0, a1, woa, wox, bo1, wo2, bo2]
    if has_res:
        args += list(res_w)
    if has_next:
        args += list(next_w)
    out_shape = [jax.ShapeDtypeStruct((N, ndo), jnp.float32)]
    if has_next:
        out_shape += [jax.ShapeDtypeStruct((N, F), jnp.float32),
                      jax.ShapeDtypeStruct((N, F), jnp.float32)]

    def body(*refs):
        _node_body(has_res, has_next, refs)

    n_row_args = 3
    in_specs = [pl.BlockSpec((_NBLK, a.shape[1]), lambda i: (i, 0))
                for a in args[:n_row_args]]
    in_specs += [pl.BlockSpec(a.shape, lambda i: (0, 0))
                 for a in args[n_row_args:]]
    out_specs = tuple(pl.BlockSpec((_NBLK, s.shape[1]), lambda i: (i, 0))
                      for s in out_shape)
    outs = pl.pallas_call(
        body,
        out_shape=tuple(out_shape),
        grid=(N // _NBLK,),
        in_specs=in_specs,
        out_specs=out_specs,
    )(*args)
    return outs


def _gather_body(ps_hbm, pd_hbm, qe_hbm, src_hbm, dst_hbm, r_hbm,
                 srcv, dstv, ga, gb, qv, sem):
    c = lax.axis_index("c")
    s = lax.axis_index("s")
    wid = s * NC + c

    pltpu.sync_copy(src_hbm.at[wid], srcv)
    pltpu.sync_copy(dst_hbm.at[wid], dstv)

    def chunk(ci, _):
        ebase = wid * EPW + ci * CH
        irow = ci * SPC
        pltpu.sync_copy(qe_hbm.at[pl.ds(ebase, CH)], qv)
        cps = []
        for j in range(SPC):
            cps.append(pltpu.async_copy(
                ps_hbm.at[srcv.at[irow + j]],
                ga.at[pl.ds(j * SUB, SUB)], sem))
            cps.append(pltpu.async_copy(
                pd_hbm.at[dstv.at[irow + j]],
                gb.at[pl.ds(j * SUB, SUB)], sem))
        for cp in cps:
            cp.wait()

        def row(rj, _):
            for col in (0, 16, 32, 48):
                a = ga[rj, pl.ds(col, 16)]
                b = gb[rj, pl.ds(col, 16)]
                q = qv[rj, pl.ds(col, 16)]
                ga[rj, pl.ds(col, 16)] = jnp.maximum(a + b + q, 0.0)
            return 0
        lax.fori_loop(0, CH, row, 0)

        pltpu.sync_copy(ga, r_hbm.at[pl.ds(ebase, CH)])
        return 0
    lax.fori_loop(0, NCHUNK, chunk, 0)


@functools.cache
def _get_gather_sc():
    mesh = plsc.VectorSubcoreMesh(core_axis_name="c", subcore_axis_name="s")
    return pl.kernel(
        _gather_body,
        out_type=jax.ShapeDtypeStruct((E, F), jnp.float32),
        mesh=mesh,
        scratch_types=[
            pltpu.VMEM((RPW, SUB), jnp.int32),
            pltpu.VMEM((RPW, SUB), jnp.int32),
            pltpu.VMEM((CH, F), jnp.float32),
            pltpu.VMEM((CH, F), jnp.float32),
            pltpu.VMEM((CH, F), jnp.float32),
            pltpu.SemaphoreType.DMA,
        ],
        compiler_params=pltpu.CompilerParams(use_tc_tiling_on_sc=False),
    )


def _gather_sc(ps, pd, qe, src, dst):
    return _get_gather_sc()(ps, pd, qe, src, dst)


def _scatter_body(eo_hbm, dst_hbm, apart_hbm, dstv, ev, zv, acc, sem):
    c = lax.axis_index("c")
    s = lax.axis_index("s")
    wid = s * NC + c

    def fill_zero(j, _):
        for col in (0, 16, 32, 48):
            zv[j, pl.ds(col, 16)] = jnp.zeros((16,), jnp.float32)
        return 0
    lax.fori_loop(0, NPT, fill_zero, 0)
    pltpu.sync_copy(zv, acc.at[pl.ds(s * NPT, NPT)])
    plsc.subcore_barrier()

    pltpu.sync_copy(dst_hbm.at[wid], dstv)

    def chunk(ci, _):
        ebase = wid * EPW + ci * CH
        irow = ci * SPC
        pltpu.sync_copy(eo_hbm.at[pl.ds(ebase, CH)], ev)
        for j in range(SPC):
            pltpu.sync_copy(ev.at[pl.ds(j * SUB, SUB)],
                            acc.at[dstv.at[irow + j]], add=True)
        return 0
    lax.fori_loop(0, NCHUNK, chunk, 0)

    plsc.subcore_barrier()
    pltpu.sync_copy(acc.at[pl.ds(s * NPT, NPT)], zv)
    pltpu.sync_copy(zv, apart_hbm.at[c, pl.ds(s * NPT, NPT)])


@functools.cache
def _get_scatter_sc():
    mesh = plsc.VectorSubcoreMesh(core_axis_name="c", subcore_axis_name="s")
    return pl.kernel(
        _scatter_body,
        out_type=jax.ShapeDtypeStruct((NC, NPAD, F), jnp.float32),
        mesh=mesh,
        scratch_types=[
            pltpu.VMEM((RPW, SUB), jnp.int32),
            pltpu.VMEM((CH, F), jnp.float32),
            pltpu.VMEM((NPT, F), jnp.float32),
            pltpu.VMEM_SHARED((NPAD, F), jnp.float32),
            pltpu.SemaphoreType.DMA,
        ],
        compiler_params=pltpu.CompilerParams(use_tc_tiling_on_sc=False),
    )


def _scatter_sc(eo, dst):
    return _get_scatter_sc()(eo, dst)


def kernel(x, edge_index, edge_attr, params):
    ar = jnp.arange(E, dtype=jnp.int32)
    perm = (ar // 2) + (ar % 2) * EP

    def sigma(ix):
        return jnp.take(ix.astype(jnp.int32), perm).reshape(NW, RPW, SUB)

    src = sigma(edge_index[0])
    dst = sigma(edge_index[1])

    L = []
    for i in range(N_LAYERS):
        lp = params["layers"][i]
        rp = params["res"][i]
        nd = [128, 64, 64][i]
        w1 = lp["rel"]["l1"]["W"]
        b1 = lp["rel"]["l1"]["b"]
        ws, wd, we = w1[:nd], w1[nd:2 * nd], w1[2 * nd:]
        w2 = lp["rel"]["l2"]["W"]
        b2 = lp["rel"]["l2"]["b"]
        wo1 = lp["obj"]["l1"]["W"]
        bo1 = lp["obj"]["l1"]["b"]
        wox, woa = wo1[:nd], wo1[nd:]
        wo2 = lp["obj"]["l2"]["W"]
        bo2 = lp["obj"]["l2"]["b"]
        L.append(dict(nd=nd, ws=ws, wd=wd, we=we, b1=b1, w2=w2, b2=b2,
                      wox=wox, woa=woa, bo1=bo1, wo2=wo2, bo2=bo2, rp=rp))

    for li in L:
        li["wsp"] = _pad_to(li["ws"], (li["nd"], F))
        li["wdp"] = _pad_to(li["wd"], (li["nd"], F))
        do = li["w2"].shape[1]
        li["w2bd"] = _bd(_pad_to(li["w2"], (F, F)))
        li["b2p"] = _dup(_pad_to(li["b2"], (F,)))
        li["webd"] = _bd(_pad_to(li["we"], (li["we"].shape[0], F)))
        li["b1p"] = _dup(_pad_to(li["b1"], (F,)))
        li["woap"] = _pad_to(li["woa"], (F, 40))

    l2 = L[2]
    wo2_last = _pad_to(l2["wo2"], (40, 8))
    bo2_last = _pad_to(l2["bo2"].reshape(1, 3), (1, 8))
    r1w_last = _pad_to(l2["rp"]["l1"]["W"], (64, 8))
    r1b_last = _pad_to(l2["rp"]["l1"]["b"].reshape(1, 3), (1, 8))
    r2w_last = _pad_to(l2["rp"]["l2"]["W"], (8, 8))
    r2b_last = _pad_to(l2["rp"]["l2"]["b"].reshape(1, 3), (1, 8))

    ps, pd = _proj0(x, L[0]["wsp"], L[0]["wdp"])
    qe_p = _qe0(edge_attr, _pad_to(L[0]["we"], (16, F)),
                _pad_to(L[0]["b1"].reshape(1, 40), (1, F)))

    xs = [x]
    eas = [edge_attr]
    xcur = x
    for i in range(N_LAYERS):
        li = L[i]
        r = _gather_sc(ps, pd, qe_p.reshape(E, F), src, dst)
        r_p = r.reshape(EP, 2 * F)
        if i < N_LAYERS - 1:
            ln = L[i + 1]
            eo_p, qe_p = _edge_tc(r_p, li["w2bd"], li["b2p"],
                                  ln["webd"], ln["b1p"])
        else:
            eo_p = _edge_tc_last(r_p, li["w2bd"], li["b2p"])
        eot = _eot(eo_p, li["w2"].shape[1])
        eo_leaf = jnp.transpose(eot[:li["w2"].shape[1]])
        apart = _scatter_sc(eo_p.reshape(E, F), dst)
        a0, a1 = apart[0, :N], apart[1, :N]
        if i < N_LAYERS - 1:
            res_w = None
            if li["rp"] is not None:
                res_w = (li["rp"]["l1"]["W"],
                         li["rp"]["l1"]["b"].reshape(1, -1),
                         li["rp"]["l2"]["W"],
                         li["rp"]["l2"]["b"].reshape(1, -1))
            xcur, ps, pd = _node_tc(
                xcur, a0, a1, li["woap"], li["wox"], li["bo1"].reshape(1, -1),
                li["wo2"], li["bo2"].reshape(1, -1),
                res_w=res_w, next_w=(L[i + 1]["wsp"], L[i + 1]["wdp"]))
        else:
            res_w = (r1w_last, r1b_last, r2w_last, r2b_last)
            (xn8,) = _node_tc(
                xcur, a0, a1, li["woap"], li["wox"], li["bo1"].reshape(1, -1),
                wo2_last, bo2_last, res_w=res_w, next_w=None)
            xcur = xn8[:, :3]
        xs.append(xcur)
        eas.append(eo_leaf)

    return (xcur, tuple(xs), tuple(eas))


# speedup vs baseline: 3.4565x; 1.0839x over previous
import functools

import jax
import jax.numpy as jnp
from jax import lax
from jax.experimental import pallas as pl
from jax.experimental.pallas import tpu as pltpu
from jax.experimental.pallas import tpu_sc as plsc

N = 10000
E = 320000
F = 64
ALPHA = 0.5
N_LAYERS = 3

NC = 2
NS = 16
NW = NC * NS
EPW = E // NW
SUB = 80
ROWS = E // SUB
RPW = ROWS // NW
CH = 400
SPC = CH // SUB
NCHUNK = EPW // CH
NPAD = 10240
NPT = NPAD // NS


def _pad_to(a, shape):
    pads = [(0, t - s) for s, t in zip(a.shape, shape)]
    return jnp.pad(a, pads)


def _bd(w):
    k, m = w.shape
    z = jnp.zeros((k, m), w.dtype)
    return jnp.concatenate(
        [jnp.concatenate([w, z], axis=1), jnp.concatenate([z, w], axis=1)],
        axis=0)


def _dup(b):
    return jnp.concatenate([b, b]).reshape(1, -1)


def _dot(a, b):
    return jax.lax.dot_general(a, b, (((1,), (0,)), ((), ())),
                               preferred_element_type=jnp.float32)


_NBLK = 2000
_PBLK = 3200
EP = E // 2


def _proj_body(x_ref, ws_ref, wd_ref, ps_ref, pd_ref):
    x = x_ref[...]
    ps_ref[...] = _dot(x, ws_ref[...])
    pd_ref[...] = _dot(x, wd_ref[...])


def _proj0(x, wsp, wdp):
    nd = x.shape[1]
    return pl.pallas_call(
        _proj_body,
        out_shape=(jax.ShapeDtypeStruct((N, F), jnp.float32),
                   jax.ShapeDtypeStruct((N, F), jnp.float32)),
        grid=(N // _NBLK,),
        in_specs=[pl.BlockSpec((_NBLK, nd), lambda i: (i, 0)),
                  pl.BlockSpec((nd, F), lambda i: (0, 0)),
                  pl.BlockSpec((nd, F), lambda i: (0, 0))],
        out_specs=(pl.BlockSpec((_NBLK, F), lambda i: (i, 0)),
                   pl.BlockSpec((_NBLK, F), lambda i: (i, 0))),
    )(x, wsp, wdp)


def _qe0_body(ea_a_ref, ea_b_ref, we_ref, b_ref, qe_ref):
    we = we_ref[...]
    b = b_ref[...]
    qa = _dot(ea_a_ref[...], we) + b
    qb = _dot(ea_b_ref[...], we) + b
    qe_ref[...] = jnp.concatenate([qa, qb], axis=1)


def _qe0(ea, wep, b1p):
    k = ea.shape[1]
    nblk = EP // _PBLK
    return pl.pallas_call(
        _qe0_body,
        out_shape=jax.ShapeDtypeStruct((EP, 2 * F), jnp.float32),
        grid=(nblk,),
        in_specs=[pl.BlockSpec((_PBLK, k), lambda i: (i, 0)),
                  pl.BlockSpec((_PBLK, k), lambda i: (i + nblk, 0)),
                  pl.BlockSpec((k, F), lambda i: (0, 0)),
                  pl.BlockSpec((1, F), lambda i: (0, 0))],
        out_specs=pl.BlockSpec((_PBLK, 2 * F), lambda i: (i, 0)),
    )(ea, ea, wep, b1p)


def _eot_body(nblk, w, eo_ref, out_ref):
    x = eo_ref[...]
    half = pl.program_id(0) // nblk
    sel = jnp.where(half == 0, x[:, :F], x[:, F:])
    out_ref[...] = sel[:, :w].T


def _eot(eo_p, width):
    w = max(8, width)
    nblk = EP // _PBLK
    return pl.pallas_call(
        functools.partial(_eot_body, nblk, w),
        out_shape=jax.ShapeDtypeStruct((w, E), jnp.float32),
        grid=(2 * nblk,),
        in_specs=[pl.BlockSpec((_PBLK, 2 * F), lambda i: (i % nblk, 0))],
        out_specs=pl.BlockSpec((w, _PBLK), lambda i: (0, i)),
    )(eo_p)


def _edge_body(r_ref, w2_ref, b2_ref, wen_ref, bn_ref, eo_ref, qn_ref):
    eo = _dot(r_ref[...], w2_ref[...]) + b2_ref[...]
    eo_ref[...] = eo
    qn_ref[...] = _dot(eo, wen_ref[...]) + bn_ref[...]


def _edge_tc(r_p, w2_bd, b2_p, wen_bd, bn_p):
    return pl.pallas_call(
        _edge_body,
        out_shape=(jax.ShapeDtypeStruct((EP, 2 * F), jnp.float32),
                   jax.ShapeDtypeStruct((EP, 2 * F), jnp.float32)),
        grid=(EP // _PBLK,),
        in_specs=[pl.BlockSpec((_PBLK, 2 * F), lambda i: (i, 0)),
                  pl.BlockSpec((2 * F, 2 * F), lambda i: (0, 0)),
                  pl.BlockSpec((1, 2 * F), lambda i: (0, 0)),
                  pl.BlockSpec((2 * F, 2 * F), lambda i: (0, 0)),
                  pl.BlockSpec((1, 2 * F), lambda i: (0, 0))],
        out_specs=(pl.BlockSpec((_PBLK, 2 * F), lambda i: (i, 0)),
                   pl.BlockSpec((_PBLK, 2 * F), lambda i: (i, 0))),
    )(r_p, w2_bd, b2_p, wen_bd, bn_p)


def _edge_last_body(r_ref, w2_ref, b2_ref, eo_ref):
    eo_ref[...] = _dot(r_ref[...], w2_ref[...]) + b2_ref[...]


def _edge_tc_last(r_p, w2_bd, b2_p):
    return pl.pallas_call(
        _edge_last_body,
        out_shape=jax.ShapeDtypeStruct((EP, 2 * F), jnp.float32),
        grid=(EP // _PBLK,),
        in_specs=[pl.BlockSpec((_PBLK, 2 * F), lambda i: (i, 0)),
                  pl.BlockSpec((2 * F, 2 * F), lambda i: (0, 0)),
                  pl.BlockSpec((1, 2 * F), lambda i: (0, 0))],
        out_specs=pl.BlockSpec((_PBLK, 2 * F), lambda i: (i, 0)),
    )(r_p, w2_bd, b2_p)


def _node_body(has_res, has_next, refs):
    (x_ref, a0_ref, a1_ref, woa_ref, wox_ref, bo1_ref,
     wo2_ref, bo2_ref) = refs[:8]
    idx = 8
    if has_res:
        r1w_ref, r1b_ref, r2w_ref, r2b_ref = refs[idx:idx + 4]
        idx += 4
    if has_next:
        wsn_ref, wdn_ref = refs[idx:idx + 2]
        idx += 2
    dep_ref = refs[idx]
    idx += 1
    outs = refs[idx:]
    x = x_ref[...] + dep_ref[0, 0] * 0.0
    agg = a0_ref[...] + a1_ref[...]
    h = _dot(x, wox_ref[...]) + _dot(agg, woa_ref[...]) + bo1_ref[...]
    h = jnp.maximum(h, 0.0)
    delta = _dot(h, wo2_ref[...]) + bo2_ref[...]
    if has_res:
        hr = jnp.maximum(_dot(x, r1w_ref[...]) + r1b_ref[...], 0.0)
        res = jnp.maximum(_dot(hr, r2w_ref[...]) + r2b_ref[...], 0.0)
    else:
        res = x
    xn = ALPHA * res + (1.0 - ALPHA) * jnp.maximum(delta, 0.0)
    outs[0][...] = xn
    if has_next:
        outs[1][...] = _dot(xn, wsn_ref[...])
        outs[2][...] = _dot(xn, wdn_ref[...])


def _node_tc(x, a0, a1, woa, wox, bo1, wo2, bo2, res_w=None, next_w=None,
             dep=None):
    ndo = wo2.shape[1]
    has_res = res_w is not None
    has_next = next_w is not None
    args = [x, a0, a1, woa, wox, bo1, wo2, bo2]
    if has_res:
        args += list(res_w)
    if has_next:
        args += list(next_w)
    if dep is None:
        dep = jnp.zeros((1, 1), jnp.float32)
    args.append(dep)
    out_shape = [jax.ShapeDtypeStruct((N, ndo), jnp.float32)]
    if has_next:
        out_shape += [jax.ShapeDtypeStruct((N, F), jnp.float32),
                      jax.ShapeDtypeStruct((N, F), jnp.float32)]

    def body(*refs):
        _node_body(has_res, has_next, refs)

    n_row_args = 3
    in_specs = [pl.BlockSpec((_NBLK, a.shape[1]), lambda i: (i, 0))
                for a in args[:n_row_args]]
    in_specs += [pl.BlockSpec(a.shape, lambda i: (0, 0))
                 for a in args[n_row_args:]]
    out_specs = tuple(pl.BlockSpec((_NBLK, s.shape[1]), lambda i: (i, 0))
                      for s in out_shape)
    outs = pl.pallas_call(
        body,
        out_shape=tuple(out_shape),
        grid=(N // _NBLK,),
        in_specs=in_specs,
        out_specs=out_specs,
    )(*args)
    return outs


def _gather_body(ps_hbm, pd_hbm, qe_hbm, src_hbm, dst_hbm, r_hbm,
                 srcv, dstv, ga, gb, qv, sem):
    c = lax.axis_index("c")
    s = lax.axis_index("s")
    wid = s * NC + c

    pltpu.sync_copy(src_hbm.at[wid], srcv)
    pltpu.sync_copy(dst_hbm.at[wid], dstv)

    def chunk(ci, _):
        ebase = wid * EPW + ci * CH
        irow = ci * SPC
        pltpu.sync_copy(qe_hbm.at[pl.ds(ebase, CH)], qv)
        cps = []
        for j in range(SPC):
            cps.append(pltpu.async_copy(
                ps_hbm.at[srcv.at[irow + j]],
                ga.at[pl.ds(j * SUB, SUB)], sem))
            cps.append(pltpu.async_copy(
                pd_hbm.at[dstv.at[irow + j]],
                gb.at[pl.ds(j * SUB, SUB)], sem))
        for cp in cps:
            cp.wait()

        def row(rj, _):
            for col in (0, 16, 32, 48):
                a = ga[rj, pl.ds(col, 16)]
                b = gb[rj, pl.ds(col, 16)]
                q = qv[rj, pl.ds(col, 16)]
                ga[rj, pl.ds(col, 16)] = jnp.maximum(a + b + q, 0.0)
            return 0
        lax.fori_loop(0, CH, row, 0)

        pltpu.sync_copy(ga, r_hbm.at[pl.ds(ebase, CH)])
        return 0
    lax.fori_loop(0, NCHUNK, chunk, 0)


@functools.cache
def _get_gather_sc():
    mesh = plsc.VectorSubcoreMesh(core_axis_name="c", subcore_axis_name="s")
    return pl.kernel(
        _gather_body,
        out_type=jax.ShapeDtypeStruct((E, F), jnp.float32),
        mesh=mesh,
        scratch_types=[
            pltpu.VMEM((RPW, SUB), jnp.int32),
            pltpu.VMEM((RPW, SUB), jnp.int32),
            pltpu.VMEM((CH, F), jnp.float32),
            pltpu.VMEM((CH, F), jnp.float32),
            pltpu.VMEM((CH, F), jnp.float32),
            pltpu.SemaphoreType.DMA,
        ],
        compiler_params=pltpu.CompilerParams(use_tc_tiling_on_sc=False),
    )


def _gather_sc(ps, pd, qe, src, dst):
    return _get_gather_sc()(ps, pd, qe, src, dst)


def _scatter_body(eo_hbm, dst_hbm, apart_hbm, dstv, ev, zv, acc, sem):
    c = lax.axis_index("c")
    s = lax.axis_index("s")
    wid = s * NC + c

    def fill_zero(j, _):
        for col in (0, 16, 32, 48):
            zv[j, pl.ds(col, 16)] = jnp.zeros((16,), jnp.float32)
        return 0
    lax.fori_loop(0, NPT, fill_zero, 0)
    pltpu.sync_copy(zv, acc.at[pl.ds(s * NPT, NPT)])
    plsc.subcore_barrier()

    pltpu.sync_copy(dst_hbm.at[wid], dstv)

    def chunk(ci, _):
        ebase = wid * EPW + ci * CH
        irow = ci * SPC
        pltpu.sync_copy(eo_hbm.at[pl.ds(ebase, CH)], ev)
        for j in range(SPC):
            pltpu.sync_copy(ev.at[pl.ds(j * SUB, SUB)],
                            acc.at[dstv.at[irow + j]], add=True)
        return 0
    lax.fori_loop(0, NCHUNK, chunk, 0)

    plsc.subcore_barrier()
    pltpu.sync_copy(acc.at[pl.ds(s * NPT, NPT)], zv)
    pltpu.sync_copy(zv, apart_hbm.at[c, pl.ds(s * NPT, NPT)])


@functools.cache
def _get_scatter_sc():
    mesh = plsc.VectorSubcoreMesh(core_axis_name="c", subcore_axis_name="s")
    return pl.kernel(
        _scatter_body,
        out_type=jax.ShapeDtypeStruct((NC, NPAD, F), jnp.float32),
        mesh=mesh,
        scratch_types=[
            pltpu.VMEM((RPW, SUB), jnp.int32),
            pltpu.VMEM((CH, F), jnp.float32),
            pltpu.VMEM((NPT, F), jnp.float32),
            pltpu.VMEM_SHARED((NPAD, F), jnp.float32),
            pltpu.SemaphoreType.DMA,
        ],
        compiler_params=pltpu.CompilerParams(use_tc_tiling_on_sc=False),
    )


def _scatter_sc(eo, dst):
    return _get_scatter_sc()(eo, dst)


def kernel(x, edge_index, edge_attr, params):
    ar = jnp.arange(E, dtype=jnp.int32)
    perm = (ar // 2) + (ar % 2) * EP

    def sigma(ix):
        return jnp.take(ix.astype(jnp.int32), perm).reshape(NW, RPW, SUB)

    src = sigma(edge_index[0])
    dst = sigma(edge_index[1])

    L = []
    for i in range(N_LAYERS):
        lp = params["layers"][i]
        rp = params["res"][i]
        nd = [128, 64, 64][i]
        w1 = lp["rel"]["l1"]["W"]
        b1 = lp["rel"]["l1"]["b"]
        ws, wd, we = w1[:nd], w1[nd:2 * nd], w1[2 * nd:]
        w2 = lp["rel"]["l2"]["W"]
        b2 = lp["rel"]["l2"]["b"]
        wo1 = lp["obj"]["l1"]["W"]
        bo1 = lp["obj"]["l1"]["b"]
        wox, woa = wo1[:nd], wo1[nd:]
        wo2 = lp["obj"]["l2"]["W"]
        bo2 = lp["obj"]["l2"]["b"]
        L.append(dict(nd=nd, ws=ws, wd=wd, we=we, b1=b1, w2=w2, b2=b2,
                      wox=wox, woa=woa, bo1=bo1, wo2=wo2, bo2=bo2, rp=rp))

    for li in L:
        li["wsp"] = _pad_to(li["ws"], (li["nd"], F))
        li["wdp"] = _pad_to(li["wd"], (li["nd"], F))
        do = li["w2"].shape[1]
        li["w2bd"] = _bd(_pad_to(li["w2"], (F, F)))
        li["b2p"] = _dup(_pad_to(li["b2"], (F,)))
        li["webd"] = _bd(_pad_to(li["we"], (li["we"].shape[0], F)))
        li["b1p"] = _dup(_pad_to(li["b1"], (F,)))
        li["woap"] = _pad_to(li["woa"], (F, 40))

    l2 = L[2]
    wo2_last = _pad_to(l2["wo2"], (40, 8))
    bo2_last = _pad_to(l2["bo2"].reshape(1, 3), (1, 8))
    r1w_last = _pad_to(l2["rp"]["l1"]["W"], (64, 8))
    r1b_last = _pad_to(l2["rp"]["l1"]["b"].reshape(1, 3), (1, 8))
    r2w_last = _pad_to(l2["rp"]["l2"]["W"], (8, 8))
    r2b_last = _pad_to(l2["rp"]["l2"]["b"].reshape(1, 3), (1, 8))

    ps, pd = _proj0(x, L[0]["wsp"], L[0]["wdp"])
    qe_p = _qe0(edge_attr, _pad_to(L[0]["we"], (16, F)),
                _pad_to(L[0]["b1"].reshape(1, 40), (1, F)))

    xs = [x]
    eas = [edge_attr]
    xcur = x
    for i in range(N_LAYERS):
        li = L[i]
        r = _gather_sc(ps, pd, qe_p.reshape(E, F), src, dst)
        r_p = r.reshape(EP, 2 * F)
        if i < N_LAYERS - 1:
            ln = L[i + 1]
            eo_p, qe_p = _edge_tc(r_p, li["w2bd"], li["b2p"],
                                  ln["webd"], ln["b1p"])
        else:
            eo_p = _edge_tc_last(r_p, li["w2bd"], li["b2p"])
        eot = _eot(eo_p, li["w2"].shape[1])
        eo_leaf = jnp.transpose(eot[:li["w2"].shape[1]])
        apart = _scatter_sc(eo_p.reshape(E, F), dst)
        a0, a1 = apart[0, :N], apart[1, :N]
        if i < N_LAYERS - 1:
            res_w = None
            if li["rp"] is not None:
                res_w = (li["rp"]["l1"]["W"],
                         li["rp"]["l1"]["b"].reshape(1, -1),
                         li["rp"]["l2"]["W"],
                         li["rp"]["l2"]["b"].reshape(1, -1))
            xcur, ps, pd = _node_tc(
                xcur, a0, a1, li["woap"], li["wox"], li["bo1"].reshape(1, -1),
                li["wo2"], li["bo2"].reshape(1, -1),
                res_w=res_w, next_w=(L[i + 1]["wsp"], L[i + 1]["wdp"]),
                dep=eot[:1, :1])
        else:
            res_w = (r1w_last, r1b_last, r2w_last, r2b_last)
            (xn8,) = _node_tc(
                xcur, a0, a1, li["woap"], li["wox"], li["bo1"].reshape(1, -1),
                wo2_last, bo2_last, res_w=res_w, next_w=None,
                dep=eot[:1, :1])
            xcur = xn8[:, :3]
        xs.append(xcur)
        eas.append(eo_leaf)

    return (xcur, tuple(xs), tuple(eas))


# speedup vs baseline: 3.6543x; 1.0572x over previous
import functools

import jax
import jax.numpy as jnp
from jax import lax
from jax.experimental import pallas as pl
from jax.experimental.pallas import tpu as pltpu
from jax.experimental.pallas import tpu_sc as plsc

N = 10000
E = 320000
F = 64
ALPHA = 0.5
N_LAYERS = 3

NC = 2
NS = 16
NW = NC * NS
EPW = E // NW
SUB = 80
ROWS = E // SUB
RPW = ROWS // NW
CH = 400
SPC = CH // SUB
NCHUNK = EPW // CH
NPAD = 10240
NPT = NPAD // NS


def _pad_to(a, shape):
    pads = [(0, t - s) for s, t in zip(a.shape, shape)]
    return jnp.pad(a, pads)


def _bd(w):
    k, m = w.shape
    z = jnp.zeros((k, m), w.dtype)
    return jnp.concatenate(
        [jnp.concatenate([w, z], axis=1), jnp.concatenate([z, w], axis=1)],
        axis=0)


def _dup(b):
    return jnp.concatenate([b, b]).reshape(1, -1)


def _dot(a, b):
    return jax.lax.dot_general(a, b, (((1,), (0,)), ((), ())),
                               preferred_element_type=jnp.float32)


_NBLK = 2000
_PBLK = 3200
EP = E // 2


def _proj_body(x_ref, ws_ref, wd_ref, ps_ref, pd_ref):
    x = x_ref[...]
    ps_ref[...] = _dot(x, ws_ref[...])
    pd_ref[...] = _dot(x, wd_ref[...])


def _proj0(x, wsp, wdp):
    nd = x.shape[1]
    return pl.pallas_call(
        _proj_body,
        out_shape=(jax.ShapeDtypeStruct((N, F), jnp.float32),
                   jax.ShapeDtypeStruct((N, F), jnp.float32)),
        grid=(N // _NBLK,),
        in_specs=[pl.BlockSpec((_NBLK, nd), lambda i: (i, 0)),
                  pl.BlockSpec((nd, F), lambda i: (0, 0)),
                  pl.BlockSpec((nd, F), lambda i: (0, 0))],
        out_specs=(pl.BlockSpec((_NBLK, F), lambda i: (i, 0)),
                   pl.BlockSpec((_NBLK, F), lambda i: (i, 0))),
    )(x, wsp, wdp)


def _qe0_body(ea_a_ref, ea_b_ref, we_ref, b_ref, qe_ref):
    we = we_ref[...]
    b = b_ref[...]
    qa = _dot(ea_a_ref[...].T, we) + b
    qb = _dot(ea_b_ref[...].T, we) + b
    qe_ref[...] = jnp.concatenate([qa, qb], axis=1)


def _qe0(ea_t, wep, b1p):
    k = ea_t.shape[0]
    nblk = EP // _PBLK
    return pl.pallas_call(
        _qe0_body,
        out_shape=jax.ShapeDtypeStruct((EP, 2 * F), jnp.float32),
        grid=(nblk,),
        in_specs=[pl.BlockSpec((k, _PBLK), lambda i: (0, i)),
                  pl.BlockSpec((k, _PBLK), lambda i: (0, i + nblk)),
                  pl.BlockSpec((k, F), lambda i: (0, 0)),
                  pl.BlockSpec((1, F), lambda i: (0, 0))],
        out_specs=pl.BlockSpec((_PBLK, 2 * F), lambda i: (i, 0)),
    )(ea_t, ea_t, wep, b1p)


def _eot_body(nblk, w, eo_ref, out_ref):
    x = eo_ref[...]
    half = pl.program_id(0) // nblk
    sel = jnp.where(half == 0, x[:, :F], x[:, F:])
    out_ref[...] = sel[:, :w].T


def _eot(eo_p, width):
    w = max(8, width)
    nblk = EP // _PBLK
    return pl.pallas_call(
        functools.partial(_eot_body, nblk, w),
        out_shape=jax.ShapeDtypeStruct((w, E), jnp.float32),
        grid=(2 * nblk,),
        in_specs=[pl.BlockSpec((_PBLK, 2 * F), lambda i: (i % nblk, 0))],
        out_specs=pl.BlockSpec((w, _PBLK), lambda i: (0, i)),
    )(eo_p)


def _edge_body(r_ref, w2_ref, b2_ref, wen_ref, bn_ref, eo_ref, qn_ref):
    eo = _dot(r_ref[...], w2_ref[...]) + b2_ref[...]
    eo_ref[...] = eo
    qn_ref[...] = _dot(eo, wen_ref[...]) + bn_ref[...]


def _edge_tc(r_p, w2_bd, b2_p, wen_bd, bn_p):
    return pl.pallas_call(
        _edge_body,
        out_shape=(jax.ShapeDtypeStruct((EP, 2 * F), jnp.float32),
                   jax.ShapeDtypeStruct((EP, 2 * F), jnp.float32)),
        grid=(EP // _PBLK,),
        in_specs=[pl.BlockSpec((_PBLK, 2 * F), lambda i: (i, 0)),
                  pl.BlockSpec((2 * F, 2 * F), lambda i: (0, 0)),
                  pl.BlockSpec((1, 2 * F), lambda i: (0, 0)),
                  pl.BlockSpec((2 * F, 2 * F), lambda i: (0, 0)),
                  pl.BlockSpec((1, 2 * F), lambda i: (0, 0))],
        out_specs=(pl.BlockSpec((_PBLK, 2 * F), lambda i: (i, 0)),
                   pl.BlockSpec((_PBLK, 2 * F), lambda i: (i, 0))),
    )(r_p, w2_bd, b2_p, wen_bd, bn_p)


def _edge_last_body(r_ref, w2_ref, b2_ref, eo_ref):
    eo_ref[...] = _dot(r_ref[...], w2_ref[...]) + b2_ref[...]


def _edge_tc_last(r_p, w2_bd, b2_p):
    return pl.pallas_call(
        _edge_last_body,
        out_shape=jax.ShapeDtypeStruct((EP, 2 * F), jnp.float32),
        grid=(EP // _PBLK,),
        in_specs=[pl.BlockSpec((_PBLK, 2 * F), lambda i: (i, 0)),
                  pl.BlockSpec((2 * F, 2 * F), lambda i: (0, 0)),
                  pl.BlockSpec((1, 2 * F), lambda i: (0, 0))],
        out_specs=pl.BlockSpec((_PBLK, 2 * F), lambda i: (i, 0)),
    )(r_p, w2_bd, b2_p)


def _node_body(has_res, has_next, refs):
    (x_ref, a0_ref, a1_ref, woa_ref, wox_ref, bo1_ref,
     wo2_ref, bo2_ref) = refs[:8]
    idx = 8
    if has_res:
        r1w_ref, r1b_ref, r2w_ref, r2b_ref = refs[idx:idx + 4]
        idx += 4
    if has_next:
        wsn_ref, wdn_ref = refs[idx:idx + 2]
        idx += 2
    dep_ref = refs[idx]
    idx += 1
    outs = refs[idx:]
    x = x_ref[...] + dep_ref[0, 0] * 0.0
    agg = a0_ref[...] + a1_ref[...]
    h = _dot(x, wox_ref[...]) + _dot(agg, woa_ref[...]) + bo1_ref[...]
    h = jnp.maximum(h, 0.0)
    delta = _dot(h, wo2_ref[...]) + bo2_ref[...]
    if has_res:
        hr = jnp.maximum(_dot(x, r1w_ref[...]) + r1b_ref[...], 0.0)
        res = jnp.maximum(_dot(hr, r2w_ref[...]) + r2b_ref[...], 0.0)
    else:
        res = x
    xn = ALPHA * res + (1.0 - ALPHA) * jnp.maximum(delta, 0.0)
    outs[0][...] = xn
    if has_next:
        outs[1][...] = _dot(xn, wsn_ref[...])
        outs[2][...] = _dot(xn, wdn_ref[...])


def _node_tc(x, a0, a1, woa, wox, bo1, wo2, bo2, res_w=None, next_w=None,
             dep=None):
    ndo = wo2.shape[1]
    has_res = res_w is not None
    has_next = next_w is not None
    args = [x, a0, a1, woa, wox, bo1, wo2, bo2]
    if has_res:
        args += list(res_w)
    if has_next:
        args += list(next_w)
    if dep is None:
        dep = jnp.zeros((1, 1), jnp.float32)
    args.append(dep)
    out_shape = [jax.ShapeDtypeStruct((N, ndo), jnp.float32)]
    if has_next:
        out_shape += [jax.ShapeDtypeStruct((N, F), jnp.float32),
                      jax.ShapeDtypeStruct((N, F), jnp.float32)]

    def body(*refs):
        _node_body(has_res, has_next, refs)

    n_row_args = 3
    in_specs = [pl.BlockSpec((_NBLK, a.shape[1]), lambda i: (i, 0))
                for a in args[:n_row_args]]
    in_specs += [pl.BlockSpec(a.shape, lambda i: (0, 0))
                 for a in args[n_row_args:]]
    out_specs = tuple(pl.BlockSpec((_NBLK, s.shape[1]), lambda i: (i, 0))
                      for s in out_shape)
    outs = pl.pallas_call(
        body,
        out_shape=tuple(out_shape),
        grid=(N // _NBLK,),
        in_specs=in_specs,
        out_specs=out_specs,
    )(*args)
    return outs


def _gather_body(ps_hbm, pd_hbm, qe_hbm, src_hbm, dst_hbm, r_hbm,
                 srcv, dstv, ga, gb, qv, sem):
    c = lax.axis_index("c")
    s = lax.axis_index("s")
    wid = s * NC + c

    pltpu.sync_copy(src_hbm.at[wid], srcv)
    pltpu.sync_copy(dst_hbm.at[wid], dstv)

    def chunk(ci, _):
        ebase = wid * EPW + ci * CH
        irow = ci * SPC
        pltpu.sync_copy(qe_hbm.at[pl.ds(ebase, CH)], qv)
        cps = []
        for j in range(SPC):
            cps.append(pltpu.async_copy(
                ps_hbm.at[srcv.at[irow + j]],
                ga.at[pl.ds(j * SUB, SUB)], sem))
            cps.append(pltpu.async_copy(
                pd_hbm.at[dstv.at[irow + j]],
                gb.at[pl.ds(j * SUB, SUB)], sem))
        for cp in cps:
            cp.wait()

        def row(rj, _):
            for col in (0, 16, 32, 48):
                a = ga[rj, pl.ds(col, 16)]
                b = gb[rj, pl.ds(col, 16)]
                q = qv[rj, pl.ds(col, 16)]
                ga[rj, pl.ds(col, 16)] = jnp.maximum(a + b + q, 0.0)
            return 0
        lax.fori_loop(0, CH, row, 0)

        pltpu.sync_copy(ga, r_hbm.at[pl.ds(ebase, CH)])
        return 0
    lax.fori_loop(0, NCHUNK, chunk, 0)


@functools.cache
def _get_gather_sc():
    mesh = plsc.VectorSubcoreMesh(core_axis_name="c", subcore_axis_name="s")
    return pl.kernel(
        _gather_body,
        out_type=jax.ShapeDtypeStruct((E, F), jnp.float32),
        mesh=mesh,
        scratch_types=[
            pltpu.VMEM((RPW, SUB), jnp.int32),
            pltpu.VMEM((RPW, SUB), jnp.int32),
            pltpu.VMEM((CH, F), jnp.float32),
            pltpu.VMEM((CH, F), jnp.float32),
            pltpu.VMEM((CH, F), jnp.float32),
            pltpu.SemaphoreType.DMA,
        ],
        compiler_params=pltpu.CompilerParams(use_tc_tiling_on_sc=False),
    )


def _gather_sc(ps, pd, qe, src, dst):
    return _get_gather_sc()(ps, pd, qe, src, dst)


def _scatter_body(eo_hbm, dst_hbm, apart_hbm, dstv, ev, zv, acc, sem):
    c = lax.axis_index("c")
    s = lax.axis_index("s")
    wid = s * NC + c

    def fill_zero(j, _):
        for col in (0, 16, 32, 48):
            zv[j, pl.ds(col, 16)] = jnp.zeros((16,), jnp.float32)
        return 0
    lax.fori_loop(0, NPT, fill_zero, 0)
    pltpu.sync_copy(zv, acc.at[pl.ds(s * NPT, NPT)])
    plsc.subcore_barrier()

    pltpu.sync_copy(dst_hbm.at[wid], dstv)

    def chunk(ci, _):
        ebase = wid * EPW + ci * CH
        irow = ci * SPC
        pltpu.sync_copy(eo_hbm.at[pl.ds(ebase, CH)], ev)
        for j in range(SPC):
            pltpu.sync_copy(ev.at[pl.ds(j * SUB, SUB)],
                            acc.at[dstv.at[irow + j]], add=True)
        return 0
    lax.fori_loop(0, NCHUNK, chunk, 0)

    plsc.subcore_barrier()
    pltpu.sync_copy(acc.at[pl.ds(s * NPT, NPT)], zv)
    pltpu.sync_copy(zv, apart_hbm.at[c, pl.ds(s * NPT, NPT)])


@functools.cache
def _get_scatter_sc():
    mesh = plsc.VectorSubcoreMesh(core_axis_name="c", subcore_axis_name="s")
    return pl.kernel(
        _scatter_body,
        out_type=jax.ShapeDtypeStruct((NC, NPAD, F), jnp.float32),
        mesh=mesh,
        scratch_types=[
            pltpu.VMEM((RPW, SUB), jnp.int32),
            pltpu.VMEM((CH, F), jnp.float32),
            pltpu.VMEM((NPT, F), jnp.float32),
            pltpu.VMEM_SHARED((NPAD, F), jnp.float32),
            pltpu.SemaphoreType.DMA,
        ],
        compiler_params=pltpu.CompilerParams(use_tc_tiling_on_sc=False),
    )


def _scatter_sc(eo, dst):
    return _get_scatter_sc()(eo, dst)


def kernel(x, edge_index, edge_attr, params):
    ar = jnp.arange(E, dtype=jnp.int32)
    perm = (ar // 2) + (ar % 2) * EP

    def sigma(ix):
        return jnp.take(ix.astype(jnp.int32), perm).reshape(NW, RPW, SUB)

    src = sigma(edge_index[0])
    dst = sigma(edge_index[1])

    L = []
    for i in range(N_LAYERS):
        lp = params["layers"][i]
        rp = params["res"][i]
        nd = [128, 64, 64][i]
        w1 = lp["rel"]["l1"]["W"]
        b1 = lp["rel"]["l1"]["b"]
        ws, wd, we = w1[:nd], w1[nd:2 * nd], w1[2 * nd:]
        w2 = lp["rel"]["l2"]["W"]
        b2 = lp["rel"]["l2"]["b"]
        wo1 = lp["obj"]["l1"]["W"]
        bo1 = lp["obj"]["l1"]["b"]
        wox, woa = wo1[:nd], wo1[nd:]
        wo2 = lp["obj"]["l2"]["W"]
        bo2 = lp["obj"]["l2"]["b"]
        L.append(dict(nd=nd, ws=ws, wd=wd, we=we, b1=b1, w2=w2, b2=b2,
                      wox=wox, woa=woa, bo1=bo1, wo2=wo2, bo2=bo2, rp=rp))

    for li in L:
        li["wsp"] = _pad_to(li["ws"], (li["nd"], F))
        li["wdp"] = _pad_to(li["wd"], (li["nd"], F))
        do = li["w2"].shape[1]
        li["w2bd"] = _bd(_pad_to(li["w2"], (F, F)))
        li["b2p"] = _dup(_pad_to(li["b2"], (F,)))
        li["webd"] = _bd(_pad_to(li["we"], (li["we"].shape[0], F)))
        li["b1p"] = _dup(_pad_to(li["b1"], (F,)))
        li["woap"] = _pad_to(li["woa"], (F, 40))

    l2 = L[2]
    wo2_last = _pad_to(l2["wo2"], (40, 8))
    bo2_last = _pad_to(l2["bo2"].reshape(1, 3), (1, 8))
    r1w_last = _pad_to(l2["rp"]["l1"]["W"], (64, 8))
    r1b_last = _pad_to(l2["rp"]["l1"]["b"].reshape(1, 3), (1, 8))
    r2w_last = _pad_to(l2["rp"]["l2"]["W"], (8, 8))
    r2b_last = _pad_to(l2["rp"]["l2"]["b"].reshape(1, 3), (1, 8))

    ps, pd = _proj0(x, L[0]["wsp"], L[0]["wdp"])
    qe_p = _qe0(jnp.transpose(edge_attr), _pad_to(L[0]["we"], (16, F)),
                _pad_to(L[0]["b1"].reshape(1, 40), (1, F)))

    xs = [x]
    eas = [edge_attr]
    xcur = x
    for i in range(N_LAYERS):
        li = L[i]
        r = _gather_sc(ps, pd, qe_p.reshape(E, F), src, dst)
        r_p = r.reshape(EP, 2 * F)
        if i < N_LAYERS - 1:
            ln = L[i + 1]
            eo_p, qe_p = _edge_tc(r_p, li["w2bd"], li["b2p"],
                                  ln["webd"], ln["b1p"])
        else:
            eo_p = _edge_tc_last(r_p, li["w2bd"], li["b2p"])
        eot = _eot(eo_p, li["w2"].shape[1])
        eo_leaf = jnp.transpose(eot[:li["w2"].shape[1]])
        apart = _scatter_sc(eo_p.reshape(E, F), dst)
        a0, a1 = apart[0, :N], apart[1, :N]
        if i < N_LAYERS - 1:
            res_w = None
            if li["rp"] is not None:
                res_w = (li["rp"]["l1"]["W"],
                         li["rp"]["l1"]["b"].reshape(1, -1),
                         li["rp"]["l2"]["W"],
                         li["rp"]["l2"]["b"].reshape(1, -1))
            xcur, ps, pd = _node_tc(
                xcur, a0, a1, li["woap"], li["wox"], li["bo1"].reshape(1, -1),
                li["wo2"], li["bo2"].reshape(1, -1),
                res_w=res_w, next_w=(L[i + 1]["wsp"], L[i + 1]["wdp"]),
                dep=eot[:1, :1])
        else:
            res_w = (r1w_last, r1b_last, r2w_last, r2b_last)
            (xn8,) = _node_tc(
                xcur, a0, a1, li["woap"], li["wox"], li["bo1"].reshape(1, -1),
                wo2_last, bo2_last, res_w=res_w, next_w=None,
                dep=eot[:1, :1])
            xcur = xn8[:, :3]
        xs.append(xcur)
        eas.append(eo_leaf)

    return (xcur, tuple(xs), tuple(eas))
